# baseline (speedup 1.0000x reference)
import jax
import jax.numpy as jnp
from jax.experimental import pallas as pl

_BS, _NA, _FH, _FW = 64, 9, 50, 84
_HH = _FH // 5


def _body(p0, p1, r0, r1, r2, r3, po, ro):
    for c, ref in enumerate((p0, p1)):
        po[:, 0, :, c, :] = jnp.swapaxes(ref[0], 0, 1)
    for c, ref in enumerate((r0, r1, r2, r3)):
        ro[:, 0, :, c, :] = jnp.swapaxes(ref[0], 0, 1)


def kernel(preds, regs):
    pin = jnp.transpose(preds, (1, 2, 0, 3))
    rin = jnp.transpose(regs, (1, 2, 0, 3))

    in_block = (1, _HH, _BS, _FW)
    p5, r5 = pl.pallas_call(
        _body,
        grid=(_NA, 5),
        in_specs=(
            [pl.BlockSpec(in_block, lambda a, ht, c=c: (c * _NA + a, ht, 0, 0))
             for c in range(2)]
            + [pl.BlockSpec(in_block, lambda a, ht, c=c: (c * _NA + a, ht, 0, 0))
               for c in range(4)]
        ),
        out_specs=[
            pl.BlockSpec((_BS, 1, _HH, 2, _FW), lambda a, ht: (0, a, ht, 0, 0)),
            pl.BlockSpec((_BS, 1, _HH, 4, _FW), lambda a, ht: (0, a, ht, 0, 0)),
        ],
        out_shape=[
            jax.ShapeDtypeStruct((_BS, _NA, _FH, 2, _FW), jnp.float32),
            jax.ShapeDtypeStruct((_BS, _NA, _FH, 4, _FW), jnp.float32),
        ],
    )(pin, pin, rin, rin, rin, rin)

    return (jnp.swapaxes(p5, 3, 4), jnp.swapaxes(r5, 3, 4))


# speedup vs baseline: 1.1897x; 1.1897x over previous
import jax
import jax.numpy as jnp
from jax.experimental import pallas as pl

_BS, _NA, _FH, _FW = 64, 9, 50, 84
_HH = _FH // 2


def _body(p0, p1, r0, r1, r2, r3, po, ro):
    for c, ref in enumerate((p0, p1)):
        po[:, 0, :, c, :] = jnp.swapaxes(ref[0], 0, 1)
    for c, ref in enumerate((r0, r1, r2, r3)):
        ro[:, 0, :, c, :] = jnp.swapaxes(ref[0], 0, 1)


def kernel(preds, regs):
    pin = jnp.transpose(preds, (1, 2, 0, 3))
    rin = jnp.transpose(regs, (1, 2, 0, 3))

    in_block = (1, _HH, _BS, _FW)
    p5, r5 = pl.pallas_call(
        _body,
        grid=(_NA, 2),
        in_specs=(
            [pl.BlockSpec(in_block, lambda a, ht, c=c: (c * _NA + a, ht, 0, 0))
             for c in range(2)]
            + [pl.BlockSpec(in_block, lambda a, ht, c=c: (c * _NA + a, ht, 0, 0))
               for c in range(4)]
        ),
        out_specs=[
            pl.BlockSpec((_BS, 1, _HH, 2, _FW), lambda a, ht: (0, a, ht, 0, 0)),
            pl.BlockSpec((_BS, 1, _HH, 4, _FW), lambda a, ht: (0, a, ht, 0, 0)),
        ],
        out_shape=[
            jax.ShapeDtypeStruct((_BS, _NA, _FH, 2, _FW), jnp.float32),
            jax.ShapeDtypeStruct((_BS, _NA, _FH, 4, _FW), jnp.float32),
        ],
    )(pin, pin, rin, rin, rin, rin)

    return (jnp.swapaxes(p5, 3, 4), jnp.swapaxes(r5, 3, 4))
